# Initial kernel scaffold; baseline (speedup 1.0000x reference)
import jax
import jax.numpy as jnp
from jax.experimental import pallas as pl


def kernel(pc, params):
    raise NotImplementedError("write your pallas kernel here")


# speedup vs baseline: 5.2404x; 5.2404x over previous
import functools

import numpy as np
import jax
import jax.numpy as jnp
from jax import lax
from jax.experimental import pallas as pl
from jax.experimental.pallas import tpu as pltpu

_BN = float(1.0 / np.sqrt(1.0 + 1e-5))
_NS = 32


def _fold(layers):
    out = []
    for (W, b, g, be) in layers:
        s = g * _BN
        out.append(((W * s[:, None]).T, (b * s + be)[None, :]))
    return out


def _cumsum_lanes(m):
    x = m
    n = m.shape[-1]
    sh = 1
    while sh < n:
        x = x + jnp.concatenate(
            [jnp.zeros(x.shape[:-1] + (sh,), x.dtype), x[..., :-sh]], axis=-1)
        sh *= 2
    return x


def _fps_body(npoint, xyz_ref, out_ref, dist_ref):
    B, _, N = xyz_ref.shape
    xyz = xyz_ref[...]
    dist_ref[...] = jnp.full((B, N), 1e10, jnp.float32)
    lane2 = lax.broadcasted_iota(jnp.int32, (B, N), 1)
    lane3 = lax.broadcasted_iota(jnp.int32, (B, 3, npoint), 2)
    out_ref[...] = jnp.zeros((B, 3, npoint), jnp.float32)

    def body(i, farthest):
        onehot = (lane2 == farthest).astype(jnp.float32)
        centroid = jnp.sum(xyz * onehot[:, None, :], axis=-1, keepdims=True)
        d = jnp.sum((xyz - centroid) ** 2, axis=1)
        dist = jnp.minimum(dist_ref[...], d)
        dist_ref[...] = dist
        out_ref[...] = jnp.where(lane3 == i, centroid, out_ref[...])
        maxv = jnp.max(dist, axis=-1, keepdims=True)
        nf = jnp.min(jnp.where(dist == maxv, lane2, N), axis=-1, keepdims=True)
        return nf

    lax.fori_loop(0, npoint, body, jnp.zeros((B, 1), jnp.int32))


def _fps(xyz_bcn, npoint):
    B, _, N = xyz_bcn.shape
    return pl.pallas_call(
        functools.partial(_fps_body, npoint),
        out_shape=jax.ShapeDtypeStruct((B, 3, npoint), jnp.float32),
        scratch_shapes=[pltpu.VMEM((B, N), jnp.float32)],
    )(xyz_bcn)


def _sa_body(S_t, r2, xyz_ref, xyzT_ref, pts_ref, nx_ref,
             w1, b1, w2, b2, w3, b3, out_ref):
    N = xyz_ref.shape[1]
    C = pts_ref.shape[2]
    Cin = C + 3
    xyz = xyz_ref[0]
    xyzT = xyzT_ref[0]
    pts = pts_ref[0]
    nx = nx_ref[0]

    sq_x = jnp.sum(xyzT * xyzT, axis=0, keepdims=True)
    sq_c = jnp.sum(nx * nx, axis=-1, keepdims=True)
    cross = lax.dot_general(nx, xyzT, (((1,), (0,)), ((), ())))
    sqd = (sq_c + sq_x) - 2.0 * cross

    mask = sqd <= r2
    m32 = mask.astype(jnp.int32)
    inc = _cumsum_lanes(m32)
    rank = inc - m32
    cnt = jnp.minimum(inc[:, N - 1:N], _NS)

    k_iota = lax.broadcasted_iota(jnp.int32, (S_t, _NS, N), 1)
    sel = (rank[:, None, :] == k_iota) & (sqd[:, None, :] <= r2)
    M = sel.astype(jnp.float32).reshape(S_t * _NS, N)

    F = jnp.concatenate([xyz, pts], axis=-1)
    g = lax.dot_general(M, F, (((1,), (0,)), ((), ())),
                        precision=lax.Precision.HIGHEST)
    cpad = jnp.concatenate([nx, jnp.zeros((S_t, C), jnp.float32)], axis=-1)
    h = (g.reshape(S_t, _NS, Cin) - cpad[:, None, :]).reshape(S_t * _NS, Cin)

    for (w, b) in ((w1, b1), (w2, b2), (w3, b3)):
        h = jnp.maximum(
            lax.dot_general(h, w[...], (((1,), (0,)), ((), ()))) + b[...], 0.0)

    C3 = h.shape[-1]
    h3 = h.reshape(S_t, _NS, C3)
    kk3 = lax.broadcasted_iota(jnp.int32, (S_t, _NS, C3), 1)
    out_ref[0] = jnp.max(jnp.where(kk3 < cnt[:, :, None], h3, -jnp.inf), axis=1)


def _sa(xyz, xyzT, pts, nxyz, layers, radius, S_t):
    B, N, _ = xyz.shape
    C = pts.shape[2]
    S = nxyz.shape[1]
    C3 = layers[-1][0].shape[1]
    in_specs = [
        pl.BlockSpec((1, N, 3), lambda b, s: (b, 0, 0)),
        pl.BlockSpec((1, 3, N), lambda b, s: (b, 0, 0)),
        pl.BlockSpec((1, N, C), lambda b, s: (b, 0, 0)),
        pl.BlockSpec((1, S_t, 3), lambda b, s: (b, s, 0)),
    ]
    args = [xyz, xyzT, pts, nxyz]
    for (w, bias) in layers:
        in_specs.append(pl.BlockSpec(w.shape, lambda b, s: (0, 0)))
        in_specs.append(pl.BlockSpec(bias.shape, lambda b, s: (0, 0)))
        args += [w, bias]
    return pl.pallas_call(
        functools.partial(_sa_body, S_t, radius * radius),
        grid=(B, S // S_t),
        in_specs=in_specs,
        out_specs=pl.BlockSpec((1, S_t, C3), lambda b, s: (b, s, 0)),
        out_shape=jax.ShapeDtypeStruct((B, S, C3), jnp.float32),
    )(*args)


def _fp_body(n2, has_p1, has_head, x1_ref, x2T_ref, p2_ref, *rest):
    out_ref = rest[-1]
    if has_p1:
        p1_ref = rest[0]
        wrefs = rest[1:-1]
    else:
        p1_ref = None
        wrefs = rest[:-1]
    x1 = x1_ref[0]
    x2T = x2T_ref[0]
    p2 = p2_ref[0]
    n1t = x1.shape[0]

    sq1 = jnp.sum(x1 * x1, axis=-1, keepdims=True)
    sq2 = jnp.sum(x2T * x2T, axis=0, keepdims=True)
    cross = lax.dot_general(x1, x2T, (((1,), (0,)), ((), ())))
    sqd = (sq1 + sq2) - 2.0 * cross

    lane = lax.broadcasted_iota(jnp.int32, (n1t, n2), 1)
    d = sqd
    wsum = jnp.zeros((n1t, 1), jnp.float32)
    Wmat = jnp.zeros((n1t, n2), jnp.float32)
    for _k in range(3):
        mk = jnp.min(d, axis=-1, keepdims=True)
        pos = jnp.min(jnp.where(d == mk, lane, n2), axis=-1, keepdims=True)
        oh = lane == pos
        rec = 1.0 / (mk + 1e-8)
        wsum = wsum + rec
        Wmat = Wmat + jnp.where(oh, rec, 0.0)
        d = jnp.where(oh, jnp.float32(jnp.inf), d)
    Wmat = Wmat / wsum

    interp = lax.dot_general(Wmat, p2, (((1,), (0,)), ((), ())),
                             precision=lax.Precision.HIGHEST)
    h = jnp.concatenate([p1_ref[0], interp], axis=-1) if has_p1 else interp

    nw = len(wrefs) // 2
    n_relu = nw - 1 if has_head else nw
    for li in range(n_relu):
        w = wrefs[2 * li][...]
        b = wrefs[2 * li + 1][...]
        h = jnp.maximum(lax.dot_general(h, w, (((1,), (0,)), ((), ()))) + b, 0.0)
    if has_head:
        w = wrefs[-2][...]
        b = wrefs[-1][...]
        logits = lax.dot_general(h, w, (((1,), (0,)), ((), ()))) + b
        m = jnp.max(logits, axis=-1, keepdims=True)
        shfted = logits - m
        out_ref[0] = shfted - jnp.log(
            jnp.sum(jnp.exp(shfted), axis=-1, keepdims=True))
    else:
        out_ref[0] = h


def _fp(x1, x2T, p1, p2, layers, head=None, n1_tile=None):
    B, n1, _ = x1.shape
    n2 = x2T.shape[2]
    C2 = p2.shape[2]
    n1t = n1_tile or n1
    in_specs = [
        pl.BlockSpec((1, n1t, 3), lambda b, s: (b, s, 0)),
        pl.BlockSpec((1, 3, n2), lambda b, s: (b, 0, 0)),
        pl.BlockSpec((1, n2, C2), lambda b, s: (b, 0, 0)),
    ]
    args = [x1, x2T, p2]
    if p1 is not None:
        in_specs.append(pl.BlockSpec((1, n1t, p1.shape[2]),
                                     lambda b, s: (b, s, 0)))
        args.append(p1)
    allw = list(layers) + (list(head) if head else [])
    for (w, bias) in allw:
        in_specs.append(pl.BlockSpec(w.shape, lambda b, s: (0, 0)))
        in_specs.append(pl.BlockSpec(bias.shape, lambda b, s: (0, 0)))
        args += [w, bias]
    Cout = allw[-1][0].shape[1]
    return pl.pallas_call(
        functools.partial(_fp_body, n2, p1 is not None, head is not None),
        grid=(B, n1 // n1t),
        in_specs=in_specs,
        out_specs=pl.BlockSpec((1, n1t, Cout), lambda b, s: (b, s, 0)),
        out_shape=jax.ShapeDtypeStruct((B, n1, Cout), jnp.float32),
    )(*args)


def kernel(pc, params):
    sa1 = _fold(params['sa1'])
    sa2 = _fold(params['sa2'])
    sa3 = _fold(params['sa3'])
    sa4 = _fold(params['sa4'])
    fp4 = _fold(params['fp4'])
    fp3 = _fold(params['fp3'])
    fp2 = _fold(params['fp2'])
    fp1 = _fold(params['fp1'])
    c1 = _fold([params['conv1']])[0]
    W2, b2 = params['conv2']
    c2 = (W2.T, b2[None, :])

    pts0 = pc.transpose(0, 2, 1)
    xyz0 = pts0[:, :, :3]
    xyz0T = pc[:, :3, :]

    nx1T = _fps(xyz0T, 1024)
    nx1 = nx1T.transpose(0, 2, 1)
    p1n = _sa(xyz0, xyz0T, pts0, nx1, sa1, 0.1, S_t=16)

    nx2T = _fps(nx1T, 256)
    nx2 = nx2T.transpose(0, 2, 1)
    p2n = _sa(nx1, nx1T, p1n, nx2, sa2, 0.2, S_t=32)

    nx3T = _fps(nx2T, 64)
    nx3 = nx3T.transpose(0, 2, 1)
    p3n = _sa(nx2, nx2T, p2n, nx3, sa3, 0.4, S_t=64)

    nx4T = _fps(nx3T, 16)
    nx4 = nx4T.transpose(0, 2, 1)
    p4n = _sa(nx3, nx3T, p3n, nx4, sa4, 0.8, S_t=16)

    l3 = _fp(nx3, nx4T, p3n, p4n, fp4)
    l2 = _fp(nx2, nx3T, p2n, l3, fp3)
    l1 = _fp(nx1, nx2T, p1n, l2, fp2)
    x = _fp(xyz0, nx1T, None, l1, fp1, head=[c1, c2],
            n1_tile=512)

    l4_points = p4n.transpose(0, 2, 1)
    return x, l4_points


# speedup vs baseline: 11.1254x; 2.1230x over previous
import functools

import numpy as np
import jax
import jax.numpy as jnp
from jax import lax
from jax.experimental import pallas as pl
from jax.experimental.pallas import tpu as pltpu

_BN = float(1.0 / np.sqrt(1.0 + 1e-5))
_NS = 32


def _fold(layers):
    out = []
    for (W, b, g, be) in layers:
        s = g * _BN
        out.append(((W * s[:, None]).T, (b * s + be)[None, :]))
    return out


def _cumsum_lanes(m):
    x = m
    n = m.shape[-1]
    sh = 1
    while sh < n:
        x = x + jnp.concatenate(
            [jnp.zeros(x.shape[:-1] + (sh,), x.dtype), x[..., :-sh]], axis=-1)
        sh *= 2
    return x


def _fps_body(npoint, xyz_ref, out_ref, dist_ref):
    B, _, N = xyz_ref.shape
    xyz = xyz_ref[...]
    dist_ref[...] = jnp.full((B, N), 1e10, jnp.float32)
    lane2 = lax.broadcasted_iota(jnp.int32, (B, N), 1)
    lane3 = lax.broadcasted_iota(jnp.int32, (B, 3, npoint), 2)
    out_ref[...] = jnp.zeros((B, 3, npoint), jnp.float32)

    def body(i, farthest):
        onehot = (lane2 == farthest).astype(jnp.float32)
        centroid = jnp.sum(xyz * onehot[:, None, :], axis=-1, keepdims=True)
        d = jnp.sum((xyz - centroid) ** 2, axis=1)
        dist = jnp.minimum(dist_ref[...], d)
        dist_ref[...] = dist
        out_ref[...] = jnp.where(lane3 == i, centroid, out_ref[...])
        maxv = jnp.max(dist, axis=-1, keepdims=True)
        nf = jnp.min(jnp.where(dist == maxv, lane2, N), axis=-1, keepdims=True)
        return nf

    lax.fori_loop(0, npoint, body, jnp.zeros((B, 1), jnp.int32))


def _fps(xyz_bcn, npoint):
    B, _, N = xyz_bcn.shape
    return pl.pallas_call(
        functools.partial(_fps_body, npoint),
        out_shape=jax.ShapeDtypeStruct((B, 3, npoint), jnp.float32),
        scratch_shapes=[pltpu.VMEM((B, N), jnp.float32)],
    )(xyz_bcn)


def _sa_body(S_t, r2, xyz_ref, xyzT_ref, pts_ref, nx_ref,
             w1, b1, w2, b2, w3, b3, out_ref):
    N = xyz_ref.shape[1]
    C = pts_ref.shape[2]
    Cin = C + 3
    xyz = xyz_ref[0]
    xyzT = xyzT_ref[0]
    pts = pts_ref[0]
    nx = nx_ref[0]

    sq_x = jnp.sum(xyzT * xyzT, axis=0, keepdims=True)
    sq_c = jnp.sum(nx * nx, axis=-1, keepdims=True)
    cross = lax.dot_general(nx, xyzT, (((1,), (0,)), ((), ())))
    sqd = (sq_c + sq_x) - 2.0 * cross

    mask = sqd <= r2
    m32 = mask.astype(jnp.int32)
    inc = _cumsum_lanes(m32)
    rank = inc - m32
    cnt = jnp.minimum(inc[:, N - 1:N], _NS)

    F = jnp.concatenate([xyz, pts], axis=-1)
    cpad = jnp.concatenate([nx, jnp.zeros((S_t, C), jnp.float32)], axis=-1)

    def group_mlp(ns_k):
        k_iota = lax.broadcasted_iota(jnp.int32, (S_t, ns_k, N), 1)
        sel = (rank[:, None, :] == k_iota) & (sqd[:, None, :] <= r2)
        M = sel.astype(jnp.float32).reshape(S_t * ns_k, N)
        g = lax.dot_general(M, F, (((1,), (0,)), ((), ())))
        h = (g.reshape(S_t, ns_k, Cin) - cpad[:, None, :]).reshape(
            S_t * ns_k, Cin)
        for (w, b) in ((w1, b1), (w2, b2), (w3, b3)):
            h = jnp.maximum(
                lax.dot_general(h, w[...], (((1,), (0,)), ((), ()))) + b[...],
                0.0)
        C3 = h.shape[-1]
        h3 = h.reshape(S_t, ns_k, C3)
        kk3 = lax.broadcasted_iota(jnp.int32, (S_t, ns_k, C3), 1)
        return jnp.max(jnp.where(kk3 < cnt[:, :, None], h3, -jnp.inf), axis=1)

    out_ref[0] = lax.cond(jnp.max(cnt) <= 8,
                          lambda: group_mlp(8), lambda: group_mlp(_NS))


def _sa(xyz, xyzT, pts, nxyz, layers, radius, S_t):
    B, N, _ = xyz.shape
    C = pts.shape[2]
    S = nxyz.shape[1]
    C3 = layers[-1][0].shape[1]
    in_specs = [
        pl.BlockSpec((1, N, 3), lambda b, s: (b, 0, 0)),
        pl.BlockSpec((1, 3, N), lambda b, s: (b, 0, 0)),
        pl.BlockSpec((1, N, C), lambda b, s: (b, 0, 0)),
        pl.BlockSpec((1, S_t, 3), lambda b, s: (b, s, 0)),
    ]
    args = [xyz, xyzT, pts, nxyz]
    for (w, bias) in layers:
        in_specs.append(pl.BlockSpec(w.shape, lambda b, s: (0, 0)))
        in_specs.append(pl.BlockSpec(bias.shape, lambda b, s: (0, 0)))
        args += [w, bias]
    return pl.pallas_call(
        functools.partial(_sa_body, S_t, radius * radius),
        grid=(B, S // S_t),
        in_specs=in_specs,
        out_specs=pl.BlockSpec((1, S_t, C3), lambda b, s: (b, s, 0)),
        out_shape=jax.ShapeDtypeStruct((B, S, C3), jnp.float32),
    )(*args)


def _fp_body(n2, has_p1, has_head, x1_ref, x2T_ref, p2_ref, *rest):
    out_ref = rest[-1]
    if has_p1:
        p1_ref = rest[0]
        wrefs = rest[1:-1]
    else:
        p1_ref = None
        wrefs = rest[:-1]
    x1 = x1_ref[0]
    x2T = x2T_ref[0]
    p2 = p2_ref[0]
    n1t = x1.shape[0]

    sq1 = jnp.sum(x1 * x1, axis=-1, keepdims=True)
    sq2 = jnp.sum(x2T * x2T, axis=0, keepdims=True)
    cross = lax.dot_general(x1, x2T, (((1,), (0,)), ((), ())))
    sqd = (sq1 + sq2) - 2.0 * cross

    lane = lax.broadcasted_iota(jnp.int32, (n1t, n2), 1)
    d = sqd
    wsum = jnp.zeros((n1t, 1), jnp.float32)
    Wmat = jnp.zeros((n1t, n2), jnp.float32)
    for _k in range(3):
        mk = jnp.min(d, axis=-1, keepdims=True)
        pos = jnp.min(jnp.where(d == mk, lane, n2), axis=-1, keepdims=True)
        oh = lane == pos
        rec = 1.0 / (mk + 1e-8)
        wsum = wsum + rec
        Wmat = Wmat + jnp.where(oh, rec, 0.0)
        d = jnp.where(oh, jnp.float32(jnp.inf), d)
    Wmat = Wmat / wsum

    interp = lax.dot_general(Wmat, p2, (((1,), (0,)), ((), ())),
                             precision=lax.Precision.HIGHEST)
    h = jnp.concatenate([p1_ref[0], interp], axis=-1) if has_p1 else interp

    nw = len(wrefs) // 2
    n_relu = nw - 1 if has_head else nw
    for li in range(n_relu):
        w = wrefs[2 * li][...]
        b = wrefs[2 * li + 1][...]
        h = jnp.maximum(lax.dot_general(h, w, (((1,), (0,)), ((), ()))) + b, 0.0)
    if has_head:
        w = wrefs[-2][...]
        b = wrefs[-1][...]
        logits = lax.dot_general(h, w, (((1,), (0,)), ((), ()))) + b
        m = jnp.max(logits, axis=-1, keepdims=True)
        shfted = logits - m
        out_ref[0] = shfted - jnp.log(
            jnp.sum(jnp.exp(shfted), axis=-1, keepdims=True))
    else:
        out_ref[0] = h


def _fp(x1, x2T, p1, p2, layers, head=None, n1_tile=None):
    B, n1, _ = x1.shape
    n2 = x2T.shape[2]
    C2 = p2.shape[2]
    n1t = n1_tile or n1
    in_specs = [
        pl.BlockSpec((1, n1t, 3), lambda b, s: (b, s, 0)),
        pl.BlockSpec((1, 3, n2), lambda b, s: (b, 0, 0)),
        pl.BlockSpec((1, n2, C2), lambda b, s: (b, 0, 0)),
    ]
    args = [x1, x2T, p2]
    if p1 is not None:
        in_specs.append(pl.BlockSpec((1, n1t, p1.shape[2]),
                                     lambda b, s: (b, s, 0)))
        args.append(p1)
    allw = list(layers) + (list(head) if head else [])
    for (w, bias) in allw:
        in_specs.append(pl.BlockSpec(w.shape, lambda b, s: (0, 0)))
        in_specs.append(pl.BlockSpec(bias.shape, lambda b, s: (0, 0)))
        args += [w, bias]
    Cout = allw[-1][0].shape[1]
    return pl.pallas_call(
        functools.partial(_fp_body, n2, p1 is not None, head is not None),
        grid=(B, n1 // n1t),
        in_specs=in_specs,
        out_specs=pl.BlockSpec((1, n1t, Cout), lambda b, s: (b, s, 0)),
        out_shape=jax.ShapeDtypeStruct((B, n1, Cout), jnp.float32),
    )(*args)


def kernel(pc, params):
    sa1 = _fold(params['sa1'])
    sa2 = _fold(params['sa2'])
    sa3 = _fold(params['sa3'])
    sa4 = _fold(params['sa4'])
    fp4 = _fold(params['fp4'])
    fp3 = _fold(params['fp3'])
    fp2 = _fold(params['fp2'])
    fp1 = _fold(params['fp1'])
    c1 = _fold([params['conv1']])[0]
    W2, b2 = params['conv2']
    c2 = (W2.T, b2[None, :])

    pts0 = pc.transpose(0, 2, 1)
    xyz0 = pts0[:, :, :3]
    xyz0T = pc[:, :3, :]

    nx1T = _fps(xyz0T, 1024)
    nx1 = nx1T.transpose(0, 2, 1)
    p1n = _sa(xyz0, xyz0T, pts0, nx1, sa1, 0.1, S_t=16)

    nx2T = _fps(nx1T, 256)
    nx2 = nx2T.transpose(0, 2, 1)
    p2n = _sa(nx1, nx1T, p1n, nx2, sa2, 0.2, S_t=32)

    nx3T = _fps(nx2T, 64)
    nx3 = nx3T.transpose(0, 2, 1)
    p3n = _sa(nx2, nx2T, p2n, nx3, sa3, 0.4, S_t=64)

    nx4T = _fps(nx3T, 16)
    nx4 = nx4T.transpose(0, 2, 1)
    p4n = _sa(nx3, nx3T, p3n, nx4, sa4, 0.8, S_t=16)

    l3 = _fp(nx3, nx4T, p3n, p4n, fp4)
    l2 = _fp(nx2, nx3T, p2n, l3, fp3)
    l1 = _fp(nx1, nx2T, p1n, l2, fp2)
    x = _fp(xyz0, nx1T, None, l1, fp1, head=[c1, c2],
            n1_tile=512)

    l4_points = p4n.transpose(0, 2, 1)
    return x, l4_points


# speedup vs baseline: 13.9752x; 1.2562x over previous
import functools

import numpy as np
import jax
import jax.numpy as jnp
from jax import lax
from jax.experimental import pallas as pl
from jax.experimental.pallas import tpu as pltpu

_BN = float(1.0 / np.sqrt(1.0 + 1e-5))
_NS = 32


def _fold(layers):
    out = []
    for (W, b, g, be) in layers:
        s = g * _BN
        out.append(((W * s[:, None]).T, (b * s + be)[None, :]))
    return out


def _cumsum_lanes(m):
    x = m
    n = m.shape[-1]
    sh = 1
    while sh < n:
        x = x + jnp.concatenate(
            [jnp.zeros(x.shape[:-1] + (sh,), x.dtype), x[..., :-sh]], axis=-1)
        sh *= 2
    return x


def _fps_body(npoint, x0_ref, x1_ref, x2_ref, out_ref, dist_ref):
    B, N = x0_ref.shape
    x0 = x0_ref[...]
    x1 = x1_ref[...]
    x2 = x2_ref[...]
    dist_ref[...] = jnp.full((B, N), 1e10, jnp.float32)
    lane2 = lax.broadcasted_iota(jnp.int32, (B, N), 1)
    lane3 = lax.broadcasted_iota(jnp.int32, (B, 3, npoint), 2)
    out_ref[...] = jnp.zeros((B, 3, npoint), jnp.float32)

    def body(i, farthest):
        onehot = (lane2 == farthest).astype(jnp.float32)
        c0 = jnp.sum(x0 * onehot, axis=-1, keepdims=True)
        c1 = jnp.sum(x1 * onehot, axis=-1, keepdims=True)
        c2 = jnp.sum(x2 * onehot, axis=-1, keepdims=True)
        d = (x0 - c0) ** 2 + (x1 - c1) ** 2 + (x2 - c2) ** 2
        dist = jnp.minimum(dist_ref[...], d)
        dist_ref[...] = dist
        c = jnp.concatenate([c0, c1, c2], axis=1)[:, :, None]
        out_ref[...] = jnp.where(lane3 == i, c, out_ref[...])
        maxv = jnp.max(dist, axis=-1, keepdims=True)
        nf = jnp.min(jnp.where(dist == maxv, lane2, N), axis=-1, keepdims=True)
        return nf

    lax.fori_loop(0, npoint, body, jnp.zeros((B, 1), jnp.int32), unroll=2)


def _fps(xyz_bcn, npoint):
    B, _, N = xyz_bcn.shape
    return pl.pallas_call(
        functools.partial(_fps_body, npoint),
        out_shape=jax.ShapeDtypeStruct((B, 3, npoint), jnp.float32),
        scratch_shapes=[pltpu.VMEM((B, N), jnp.float32)],
    )(xyz_bcn[:, 0, :], xyz_bcn[:, 1, :], xyz_bcn[:, 2, :])


def _sa_body(S_t, r2, xyz_ref, xyzT_ref, pts_ref, nx_ref,
             w1, b1, w2, b2, w3, b3, out_ref):
    N = xyz_ref.shape[1]
    C = pts_ref.shape[2]
    Cin = C + 3
    xyz = xyz_ref[0]
    xyzT = xyzT_ref[0]
    pts = pts_ref[0]
    nx = nx_ref[0]

    sq_x = jnp.sum(xyzT * xyzT, axis=0, keepdims=True)
    sq_c = jnp.sum(nx * nx, axis=-1, keepdims=True)
    cross = lax.dot_general(nx, xyzT, (((1,), (0,)), ((), ())))
    sqd = (sq_c + sq_x) - 2.0 * cross

    mask = sqd <= r2
    m32 = mask.astype(jnp.int32)
    inc = _cumsum_lanes(m32)
    rank = inc - m32
    cnt = jnp.minimum(inc[:, N - 1:N], _NS)

    F = jnp.concatenate([xyz, pts], axis=-1)
    cpad = jnp.concatenate([nx, jnp.zeros((S_t, C), jnp.float32)], axis=-1)

    def group_mlp(ns_k):
        k_iota = lax.broadcasted_iota(jnp.int32, (S_t, ns_k, N), 1)
        sel = (rank[:, None, :] == k_iota) & (sqd[:, None, :] <= r2)
        M = sel.astype(jnp.float32).reshape(S_t * ns_k, N)
        g = lax.dot_general(M, F, (((1,), (0,)), ((), ())))
        h = (g.reshape(S_t, ns_k, Cin) - cpad[:, None, :]).reshape(
            S_t * ns_k, Cin)
        for (w, b) in ((w1, b1), (w2, b2), (w3, b3)):
            h = jnp.maximum(
                lax.dot_general(h, w[...], (((1,), (0,)), ((), ()))) + b[...],
                0.0)
        C3 = h.shape[-1]
        h3 = h.reshape(S_t, ns_k, C3)
        kk3 = lax.broadcasted_iota(jnp.int32, (S_t, ns_k, C3), 1)
        return jnp.max(jnp.where(kk3 < cnt[:, :, None], h3, -jnp.inf), axis=1)

    out_ref[0] = lax.cond(jnp.max(cnt) <= 8,
                          lambda: group_mlp(8), lambda: group_mlp(_NS))


def _sa(xyz, xyzT, pts, nxyz, layers, radius, S_t):
    B, N, _ = xyz.shape
    C = pts.shape[2]
    S = nxyz.shape[1]
    C3 = layers[-1][0].shape[1]
    in_specs = [
        pl.BlockSpec((1, N, 3), lambda b, s: (b, 0, 0)),
        pl.BlockSpec((1, 3, N), lambda b, s: (b, 0, 0)),
        pl.BlockSpec((1, N, C), lambda b, s: (b, 0, 0)),
        pl.BlockSpec((1, S_t, 3), lambda b, s: (b, s, 0)),
    ]
    args = [xyz, xyzT, pts, nxyz]
    for (w, bias) in layers:
        in_specs.append(pl.BlockSpec(w.shape, lambda b, s: (0, 0)))
        in_specs.append(pl.BlockSpec(bias.shape, lambda b, s: (0, 0)))
        args += [w, bias]
    return pl.pallas_call(
        functools.partial(_sa_body, S_t, radius * radius),
        grid=(B, S // S_t),
        in_specs=in_specs,
        out_specs=pl.BlockSpec((1, S_t, C3), lambda b, s: (b, s, 0)),
        out_shape=jax.ShapeDtypeStruct((B, S, C3), jnp.float32),
    )(*args)


def _fp_body(n2, has_p1, has_head, x1_ref, x2T_ref, p2_ref, *rest):
    out_ref = rest[-1]
    if has_p1:
        p1_ref = rest[0]
        wrefs = rest[1:-1]
    else:
        p1_ref = None
        wrefs = rest[:-1]
    x1 = x1_ref[0]
    x2T = x2T_ref[0]
    p2 = p2_ref[0]
    n1t = x1.shape[0]

    sq1 = jnp.sum(x1 * x1, axis=-1, keepdims=True)
    sq2 = jnp.sum(x2T * x2T, axis=0, keepdims=True)
    cross = lax.dot_general(x1, x2T, (((1,), (0,)), ((), ())))
    sqd = (sq1 + sq2) - 2.0 * cross

    lane = lax.broadcasted_iota(jnp.int32, (n1t, n2), 1)
    d = sqd
    wsum = jnp.zeros((n1t, 1), jnp.float32)
    Wmat = jnp.zeros((n1t, n2), jnp.float32)
    for _k in range(3):
        mk = jnp.min(d, axis=-1, keepdims=True)
        pos = jnp.min(jnp.where(d == mk, lane, n2), axis=-1, keepdims=True)
        oh = lane == pos
        rec = 1.0 / (mk + 1e-8)
        wsum = wsum + rec
        Wmat = Wmat + jnp.where(oh, rec, 0.0)
        d = jnp.where(oh, jnp.float32(jnp.inf), d)
    Wmat = Wmat / wsum

    interp = lax.dot_general(Wmat, p2, (((1,), (0,)), ((), ())),
                             precision=lax.Precision.HIGHEST)
    h = jnp.concatenate([p1_ref[0], interp], axis=-1) if has_p1 else interp

    nw = len(wrefs) // 2
    n_relu = nw - 1 if has_head else nw
    for li in range(n_relu):
        w = wrefs[2 * li][...]
        b = wrefs[2 * li + 1][...]
        h = jnp.maximum(lax.dot_general(h, w, (((1,), (0,)), ((), ()))) + b, 0.0)
    if has_head:
        w = wrefs[-2][...]
        b = wrefs[-1][...]
        logits = lax.dot_general(h, w, (((1,), (0,)), ((), ()))) + b
        m = jnp.max(logits, axis=-1, keepdims=True)
        shfted = logits - m
        out_ref[0] = shfted - jnp.log(
            jnp.sum(jnp.exp(shfted), axis=-1, keepdims=True))
    else:
        out_ref[0] = h


def _fp(x1, x2T, p1, p2, layers, head=None, n1_tile=None):
    B, n1, _ = x1.shape
    n2 = x2T.shape[2]
    C2 = p2.shape[2]
    n1t = n1_tile or n1
    in_specs = [
        pl.BlockSpec((1, n1t, 3), lambda b, s: (b, s, 0)),
        pl.BlockSpec((1, 3, n2), lambda b, s: (b, 0, 0)),
        pl.BlockSpec((1, n2, C2), lambda b, s: (b, 0, 0)),
    ]
    args = [x1, x2T, p2]
    if p1 is not None:
        in_specs.append(pl.BlockSpec((1, n1t, p1.shape[2]),
                                     lambda b, s: (b, s, 0)))
        args.append(p1)
    allw = list(layers) + (list(head) if head else [])
    for (w, bias) in allw:
        in_specs.append(pl.BlockSpec(w.shape, lambda b, s: (0, 0)))
        in_specs.append(pl.BlockSpec(bias.shape, lambda b, s: (0, 0)))
        args += [w, bias]
    Cout = allw[-1][0].shape[1]
    return pl.pallas_call(
        functools.partial(_fp_body, n2, p1 is not None, head is not None),
        grid=(B, n1 // n1t),
        in_specs=in_specs,
        out_specs=pl.BlockSpec((1, n1t, Cout), lambda b, s: (b, s, 0)),
        out_shape=jax.ShapeDtypeStruct((B, n1, Cout), jnp.float32),
    )(*args)


def kernel(pc, params):
    sa1 = _fold(params['sa1'])
    sa2 = _fold(params['sa2'])
    sa3 = _fold(params['sa3'])
    sa4 = _fold(params['sa4'])
    fp4 = _fold(params['fp4'])
    fp3 = _fold(params['fp3'])
    fp2 = _fold(params['fp2'])
    fp1 = _fold(params['fp1'])
    c1 = _fold([params['conv1']])[0]
    W2, b2 = params['conv2']
    c2 = (W2.T, b2[None, :])

    pts0 = pc.transpose(0, 2, 1)
    xyz0 = pts0[:, :, :3]
    xyz0T = pc[:, :3, :]

    nx1T = _fps(xyz0T, 1024)
    nx1 = nx1T.transpose(0, 2, 1)
    p1n = _sa(xyz0, xyz0T, pts0, nx1, sa1, 0.1, S_t=32)

    nx2T = _fps(nx1T, 256)
    nx2 = nx2T.transpose(0, 2, 1)
    p2n = _sa(nx1, nx1T, p1n, nx2, sa2, 0.2, S_t=32)

    nx3T = _fps(nx2T, 64)
    nx3 = nx3T.transpose(0, 2, 1)
    p3n = _sa(nx2, nx2T, p2n, nx3, sa3, 0.4, S_t=64)

    nx4T = _fps(nx3T, 16)
    nx4 = nx4T.transpose(0, 2, 1)
    p4n = _sa(nx3, nx3T, p3n, nx4, sa4, 0.8, S_t=16)

    l3 = _fp(nx3, nx4T, p3n, p4n, fp4)
    l2 = _fp(nx2, nx3T, p2n, l3, fp3)
    l1 = _fp(nx1, nx2T, p1n, l2, fp2)
    x = _fp(xyz0, nx1T, None, l1, fp1, head=[c1, c2],
            n1_tile=512)

    l4_points = p4n.transpose(0, 2, 1)
    return x, l4_points
